# Initial kernel scaffold; baseline (speedup 1.0000x reference)
import jax
import jax.numpy as jnp
from jax.experimental import pallas as pl


def kernel(x, bases, slopes):
    raise NotImplementedError("write your pallas kernel here")


# speedup vs baseline: 278.0891x; 278.0891x over previous
import functools

import jax
import jax.numpy as jnp
import numpy as np
from jax import lax
from jax.experimental import pallas as pl
from jax.experimental.pallas import tpu as pltpu
from jax.experimental.pallas import tpu_sc as plsc

_GRID_SIZE = 1024
_INPUT_MIN = -1.0
_INPUT_RANGE = 2.0
_CLIP_HI = np.float32(1.0 - 1e-06)

_ROWS = 16384
_COLS = 100
_N = _ROWS * _COLS

_NC = 2
_NS = 16
_NW = _NC * _NS
_CHUNK = _N // _NW
_LANES = 16
_VECS = _CHUNK // _LANES


def _spline_body(x_hbm, bases_hbm, slopes_hbm, out_hbm, xbuf, ybuf, btab, stab):
    wid = lax.axis_index("s") * _NC + lax.axis_index("c")
    base = wid * _CHUNK

    pltpu.sync_copy(bases_hbm, btab)
    pltpu.sync_copy(slopes_hbm, stab)
    pltpu.sync_copy(x_hbm.at[pl.ds(base, _CHUNK)], xbuf)

    def step(i, carry):
        off = i * _LANES
        xv = xbuf[pl.ds(off, _LANES)]
        xn = (xv - _INPUT_MIN) * (1.0 / _INPUT_RANGE)
        xn = jnp.minimum(jnp.maximum(xn, 0.0), _CLIP_HI)
        t = xn * float(_GRID_SIZE)
        idx = t.astype(jnp.int32)
        xl = t - idx.astype(jnp.float32)
        b = plsc.load_gather(btab, [idx])
        s = plsc.load_gather(stab, [idx])
        ybuf[pl.ds(off, _LANES)] = b + s * xl
        return carry

    lax.fori_loop(0, _VECS, step, 0)

    pltpu.sync_copy(ybuf, out_hbm.at[pl.ds(base, _CHUNK)])


@functools.partial(jax.jit, static_argnames=())
def kernel(x, bases, slopes):
    mesh = plsc.VectorSubcoreMesh(core_axis_name="c", subcore_axis_name="s")
    run = pl.kernel(
        _spline_body,
        mesh=mesh,
        out_type=jax.ShapeDtypeStruct((_N,), jnp.float32),
        scratch_types=[
            pltpu.VMEM((_CHUNK,), jnp.float32),
            pltpu.VMEM((_CHUNK,), jnp.float32),
            pltpu.VMEM((_GRID_SIZE,), jnp.float32),
            pltpu.VMEM((_GRID_SIZE,), jnp.float32),
        ],
        compiler_params=pltpu.CompilerParams(needs_layout_passes=False),
    )
    y = run(x.reshape(_N), bases, slopes)
    return y.reshape(_ROWS, _COLS)


# speedup vs baseline: 501.8558x; 1.8047x over previous
import functools

import jax
import jax.numpy as jnp
import numpy as np
from jax import lax
from jax.experimental import pallas as pl
from jax.experimental.pallas import tpu as pltpu
from jax.experimental.pallas import tpu_sc as plsc

_GRID_SIZE = 1024
_INPUT_MIN = -1.0
_INPUT_RANGE = 2.0
_CLIP_HI = np.float32(1.0 - 1e-06)

_ROWS = 16384
_COLS = 100
_LANES = 16

_NC = 2
_NS = 16
_NW = _NC * _NS
_ROWS_PER_TILE = _ROWS // _NW

_OFFS_DISJOINT = (0, 16, 32, 48, 64)
_OFFS_TAIL = (80, _COLS - _LANES)


def _spline_body(x_hbm, bases_hbm, slopes_hbm, out_hbm, xbuf, btab, stab):
    wid = lax.axis_index("s") * _NC + lax.axis_index("c")
    r0 = wid * _ROWS_PER_TILE

    pltpu.sync_copy(bases_hbm, btab)
    pltpu.sync_copy(slopes_hbm, stab)
    pltpu.sync_copy(x_hbm.at[pl.ds(r0, _ROWS_PER_TILE), :], xbuf)

    def _interp(xv):
        xn = (xv - _INPUT_MIN) * (1.0 / _INPUT_RANGE)
        xn = jnp.minimum(jnp.maximum(xn, 0.0), _CLIP_HI)
        t = xn * float(_GRID_SIZE)
        idx = t.astype(jnp.int32)
        xl = t - idx.astype(jnp.float32)
        b = plsc.load_gather(btab, [idx])
        s = plsc.load_gather(stab, [idx])
        return b + s * xl

    @plsc.parallel_loop(0, _ROWS_PER_TILE, unroll=2)
    def _row(i):
        tails = [xbuf[i, pl.ds(off, _LANES)] for off in _OFFS_TAIL]
        for off in _OFFS_DISJOINT:
            xbuf[i, pl.ds(off, _LANES)] = _interp(xbuf[i, pl.ds(off, _LANES)])
        for off, xv in zip(_OFFS_TAIL, tails):
            xbuf[i, pl.ds(off, _LANES)] = _interp(xv)

    pltpu.sync_copy(xbuf, out_hbm.at[pl.ds(r0, _ROWS_PER_TILE), :])


@jax.jit
def kernel(x, bases, slopes):
    mesh = plsc.VectorSubcoreMesh(core_axis_name="c", subcore_axis_name="s")
    run = pl.kernel(
        _spline_body,
        mesh=mesh,
        out_type=jax.ShapeDtypeStruct((_ROWS, _COLS), jnp.float32),
        scratch_types=[
            pltpu.VMEM((_ROWS_PER_TILE, _COLS), jnp.float32),
            pltpu.VMEM((_GRID_SIZE,), jnp.float32),
            pltpu.VMEM((_GRID_SIZE,), jnp.float32),
        ],
        compiler_params=pltpu.CompilerParams(needs_layout_passes=False),
    )
    return run(x, bases, slopes)
